# baseline (speedup 1.0000x reference)
import functools

import jax
import jax.numpy as jnp
from jax import lax
from jax.experimental import pallas as pl
from jax.experimental.pallas import tpu as pltpu
from jax.experimental.pallas import tpu_sc as plsc

_IDXW = 128
_CHUNK = 512


@functools.lru_cache(maxsize=None)
def _build(n, d, h):
    info = plsc.get_sparse_core_info()
    nc, ns, lanes = info.num_cores, info.num_subcores, info.num_lanes
    nw = nc * ns
    assert h == lanes and _CHUNK % lanes == 0 and _IDXW % lanes == 0
    per_w = n // nw
    assert per_w % _CHUNK == 0
    n_chunks = per_w // _CHUNK
    spc = _CHUNK // _IDXW
    rows_pw = per_w // _IDXW

    mesh = plsc.VectorSubcoreMesh(core_axis_name="c", subcore_axis_name="s")

    @functools.partial(
        pl.kernel,
        mesh=mesh,
        compiler_params=pltpu.CompilerParams(use_tc_tiling_on_sc=False),
        out_type=jax.ShapeDtypeStruct((n, d), jnp.float32),
        scratch_types=[
            pltpu.VMEM((2, spc, _IDXW), jnp.int32),
            pltpu.VMEM((2, _CHUNK, d), jnp.float32),
            pltpu.VMEM((lanes,), jnp.int32),
            pltpu.SemaphoreType.DMA,
            pltpu.SemaphoreType.DMA,
        ],
    )
    def gather(ids_hbm, offs_hbm, table_hbm, out_hbm, idx_v, rows_v, offs_v,
               gsem, wsem):
        wid = lax.axis_index("s") * nc + lax.axis_index("c")
        idx_row0 = wid * rows_pw
        out_row0 = wid * per_w
        pltpu.sync_copy(offs_hbm, offs_v)
        off = offs_v[...]

        def load_shift(c, slot):
            pltpu.sync_copy(ids_hbm.at[pl.ds(idx_row0 + c * spc, spc)],
                            idx_v.at[slot])
            for j in range(spc):
                @pl.loop(0, _IDXW // lanes)
                def _(i):
                    s = pl.ds(i * lanes, lanes)
                    idx_v[slot, j, s] = idx_v[slot, j, s] + off

        def fire_gathers(slot):
            return [
                pltpu.async_copy(table_hbm.at[idx_v.at[slot, j]],
                                 rows_v.at[slot, pl.ds(j * _IDXW, _IDXW)],
                                 gsem)
                for j in range(spc)
            ]

        load_shift(0, 0)
        g = fire_gathers(0)
        pending_w = []
        for c in range(n_chunks):
            slot = c & 1
            if c + 1 < n_chunks:
                load_shift(c + 1, 1 - slot)
                if pending_w:
                    pending_w.pop(0).wait()
                g_next = fire_gathers(1 - slot)
            for hnd in g:
                hnd.wait()
            pending_w.append(pltpu.async_copy(
                rows_v.at[slot],
                out_hbm.at[pl.ds(out_row0 + c * _CHUNK, _CHUNK)],
                wsem))
            if c + 1 < n_chunks:
                g = g_next
        for w in pending_w:
            w.wait()

    return gather


def kernel(input_ids, offsets, table):
    b, t, h = input_ids.shape
    d = table.shape[1]
    n = b * t * h
    ids2d = input_ids.reshape(n // _IDXW, _IDXW)
    out = _build(n, d, h)(ids2d, offsets, table)
    return out.reshape(b, t, h * d)


# speedup vs baseline: 1.0183x; 1.0183x over previous
import functools

import jax
import jax.numpy as jnp
from jax import lax
from jax.experimental import pallas as pl
from jax.experimental.pallas import tpu as pltpu
from jax.experimental.pallas import tpu_sc as plsc

_IDXW = 128
_CHUNK = 512


@functools.lru_cache(maxsize=None)
def _build(n, d, h):
    info = plsc.get_sparse_core_info()
    nc, ns, lanes = info.num_cores, info.num_subcores, info.num_lanes
    nw = nc * ns
    assert h == lanes and _CHUNK % lanes == 0 and _IDXW % lanes == 0
    per_w = n // nw
    assert per_w % _CHUNK == 0
    n_chunks = per_w // _CHUNK
    spc = _CHUNK // _IDXW
    rows_pw = per_w // _IDXW

    mesh = plsc.VectorSubcoreMesh(core_axis_name="c", subcore_axis_name="s")

    @functools.partial(
        pl.kernel,
        mesh=mesh,
        compiler_params=pltpu.CompilerParams(use_tc_tiling_on_sc=False),
        out_type=jax.ShapeDtypeStruct((n, d), jnp.float32),
        scratch_types=[
            pltpu.VMEM((2, spc, _IDXW), jnp.int32),
            pltpu.VMEM((2, _CHUNK, d), jnp.float32),
            pltpu.VMEM((lanes,), jnp.int32),
            pltpu.SemaphoreType.DMA,
            pltpu.SemaphoreType.DMA,
        ],
    )
    def gather(ids_hbm, offs_hbm, table_hbm, out_hbm, idx_v, rows_v, offs_v,
               gsem, wsem):
        wid = lax.axis_index("s") * nc + lax.axis_index("c")
        idx_row0 = wid * rows_pw
        out_row0 = wid * per_w
        pltpu.sync_copy(offs_hbm, offs_v)
        off = offs_v[...]

        def load_shift(c, slot):
            pltpu.sync_copy(ids_hbm.at[pl.ds(idx_row0 + c * spc, spc)],
                            idx_v.at[slot])
            for j in range(spc):
                @pl.loop(0, _IDXW // lanes)
                def _(i):
                    s = pl.ds(i * lanes, lanes)
                    idx_v[slot, j, s] = idx_v[slot, j, s] + off

        def fire_gathers(slot):
            return [
                pltpu.async_copy(table_hbm.at[idx_v.at[slot, j]],
                                 rows_v.at[slot, pl.ds(j * _IDXW, _IDXW)],
                                 gsem)
                for j in range(spc)
            ]

        load_shift(0, 0)
        g = fire_gathers(0)
        pending_w = []
        for c in range(n_chunks):
            slot = c & 1
            if c + 1 < n_chunks:
                load_shift(c + 1, 1 - slot)
                if pending_w:
                    pending_w.pop(0).wait()
                g_next = fire_gathers(1 - slot)
            for hnd in g:
                hnd.wait()
            pending_w.append(pltpu.async_copy(
                rows_v.at[slot],
                out_hbm.at[pl.ds(out_row0 + c * _CHUNK, _CHUNK)],
                wsem))
            if c + 1 < n_chunks:
                g = g_next
        for w in pending_w:
            w.wait()

    return gather


@functools.lru_cache(maxsize=None)
def _build_tiler(n_rows, n_cols, rows_per_step):
    info = plsc.get_sparse_core_info()
    nw = info.num_cores * info.num_subcores
    nc = info.num_cores
    assert n_rows % (nw * rows_per_step) == 0 and n_cols % 128 == 0
    steps = n_rows // (nw * rows_per_step)
    per_w = n_rows // nw
    cpr = n_cols // 128
    in_rps = rows_per_step * cpr

    mesh = plsc.VectorSubcoreMesh(core_axis_name="c", subcore_axis_name="s")

    @functools.partial(
        pl.kernel,
        mesh=mesh,
        compiler_params=pltpu.CompilerParams(use_tc_tiling_on_sc=True),
        out_type=jax.ShapeDtypeStruct((n_rows, n_cols), jnp.float32),
        scratch_types=[
            pltpu.VMEM((2, in_rps, 128), jnp.float32),
            pltpu.SemaphoreType.DMA,
            pltpu.SemaphoreType.DMA,
        ],
    )
    def tile_out(in_hbm, out_hbm, buf, rsem, wsem):
        wid = lax.axis_index("s") * nc + lax.axis_index("c")
        row0 = wid * per_w

        def read(c, slot):
            return pltpu.async_copy(
                in_hbm.at[pl.ds((row0 + c * rows_per_step) * cpr, in_rps), :],
                buf.at[slot], rsem)

        r = read(0, 0)
        pending_w = []
        for c in range(steps):
            slot = c & 1
            if c + 1 < steps:
                if pending_w:
                    pending_w.pop(0).wait()
                r_next = read(c + 1, 1 - slot)
            r.wait()
            pending_w.append(pltpu.async_copy(
                buf.at[slot].reshape(rows_per_step, n_cols),
                out_hbm.at[pl.ds((row0 + c * rows_per_step), rows_per_step), :],
                wsem))
            if c + 1 < steps:
                r = r_next
        for w in pending_w:
            w.wait()

    return tile_out


def kernel(input_ids, offsets, table):
    b, t, h = input_ids.shape
    d = table.shape[1]
    n = b * t * h
    ids2d = input_ids.reshape(n // _IDXW, _IDXW)
    out = _build(n, d, h)(ids2d, offsets, table)
    out_tiled = _build_tiler(b * t, h * d, 16)(out.reshape(n * d // 128, 128))
    return out_tiled.reshape(b, t, h * d)


# speedup vs baseline: 1.0217x; 1.0034x over previous
import functools

import jax
import jax.numpy as jnp
from jax import lax
from jax.experimental import pallas as pl
from jax.experimental.pallas import tpu as pltpu
from jax.experimental.pallas import tpu_sc as plsc

_IDXW = 128
_CHUNK = 512


@functools.lru_cache(maxsize=None)
def _build(n, d, h):
    info = plsc.get_sparse_core_info()
    nc, ns, lanes = info.num_cores, info.num_subcores, info.num_lanes
    nw = nc * ns
    assert h == lanes and _CHUNK % lanes == 0
    per_w = n // nw
    assert per_w % _CHUNK == 0
    n_chunks = per_w // _CHUNK

    mesh = plsc.VectorSubcoreMesh(core_axis_name="c", subcore_axis_name="s")

    @functools.partial(
        pl.kernel,
        mesh=mesh,
        compiler_params=pltpu.CompilerParams(use_tc_tiling_on_sc=False),
        out_type=jax.ShapeDtypeStruct((n, d), jnp.float32),
        scratch_types=[
            pltpu.VMEM((2, _CHUNK), jnp.int32),
            pltpu.VMEM((2, _CHUNK, d), jnp.float32),
            pltpu.VMEM((lanes,), jnp.int32),
            pltpu.SemaphoreType.DMA,
            pltpu.SemaphoreType.DMA,
        ],
    )
    def gather(ids_hbm, offs_hbm, table_hbm, out_hbm, idx_v, rows_v, offs_v,
               gsem, wsem):
        wid = lax.axis_index("s") * nc + lax.axis_index("c")
        row0 = wid * per_w
        pltpu.sync_copy(offs_hbm, offs_v)
        off = offs_v[...]

        def load_shift(c, slot):
            pltpu.sync_copy(ids_hbm.at[pl.ds(row0 + c * _CHUNK, _CHUNK)],
                            idx_v.at[slot])

            @pl.loop(0, _CHUNK // lanes)
            def _(i):
                s = pl.ds(i * lanes, lanes)
                idx_v[slot, s] = idx_v[slot, s] + off

        def fire_gather(slot):
            return pltpu.async_copy(table_hbm.at[idx_v.at[slot]],
                                    rows_v.at[slot], gsem)

        load_shift(0, 0)
        g = fire_gather(0)
        pending_w = []
        for c in range(n_chunks):
            slot = c & 1
            if c + 1 < n_chunks:
                load_shift(c + 1, 1 - slot)
                if pending_w:
                    pending_w.pop(0).wait()
                g_next = fire_gather(1 - slot)
            g.wait()
            pending_w.append(pltpu.async_copy(
                rows_v.at[slot],
                out_hbm.at[pl.ds(row0 + c * _CHUNK, _CHUNK)],
                wsem))
            if c + 1 < n_chunks:
                g = g_next
        for w in pending_w:
            w.wait()

    return gather


@functools.lru_cache(maxsize=None)
def _build_tiler(n_rows, n_cols, rows_per_step):
    info = plsc.get_sparse_core_info()
    nw = info.num_cores * info.num_subcores
    nc = info.num_cores
    assert n_rows % (nw * rows_per_step) == 0 and n_cols % 128 == 0
    steps = n_rows // (nw * rows_per_step)
    per_w = n_rows // nw
    cpr = n_cols // 128
    in_rps = rows_per_step * cpr

    mesh = plsc.VectorSubcoreMesh(core_axis_name="c", subcore_axis_name="s")

    @functools.partial(
        pl.kernel,
        mesh=mesh,
        compiler_params=pltpu.CompilerParams(use_tc_tiling_on_sc=True),
        out_type=jax.ShapeDtypeStruct((n_rows, n_cols), jnp.float32),
        scratch_types=[
            pltpu.VMEM((2, in_rps, 128), jnp.float32),
            pltpu.SemaphoreType.DMA,
            pltpu.SemaphoreType.DMA,
        ],
    )
    def tile_out(in_hbm, out_hbm, buf, rsem, wsem):
        wid = lax.axis_index("s") * nc + lax.axis_index("c")
        row0 = wid * per_w

        def read(c, slot):
            return pltpu.async_copy(
                in_hbm.at[pl.ds((row0 + c * rows_per_step) * cpr, in_rps), :],
                buf.at[slot], rsem)

        r = read(0, 0)
        pending_w = []
        for c in range(steps):
            slot = c & 1
            if c + 1 < steps:
                if pending_w:
                    pending_w.pop(0).wait()
                r_next = read(c + 1, 1 - slot)
            r.wait()
            pending_w.append(pltpu.async_copy(
                buf.at[slot].reshape(rows_per_step, n_cols),
                out_hbm.at[pl.ds((row0 + c * rows_per_step), rows_per_step), :],
                wsem))
            if c + 1 < steps:
                r = r_next
        for w in pending_w:
            w.wait()

    return tile_out


def kernel(input_ids, offsets, table):
    b, t, h = input_ids.shape
    d = table.shape[1]
    n = b * t * h
    ids1d = input_ids.reshape(n)
    out = _build(n, d, h)(ids1d, offsets, table)
    out_tiled = _build_tiler(b * t, h * d, 16)(out.reshape(n * d // 128, 128))

    return out_tiled.reshape(b, t, h * d)


# speedup vs baseline: 1.0249x; 1.0031x over previous
import functools

import jax
import jax.numpy as jnp
from jax import lax
from jax.experimental import pallas as pl
from jax.experimental.pallas import tpu as pltpu
from jax.experimental.pallas import tpu_sc as plsc

_IDXW = 128
_CHUNK = 512


@functools.lru_cache(maxsize=None)
def _build(b, t, h, d):
    info = plsc.get_sparse_core_info()
    nc, ns, lanes = info.num_cores, info.num_subcores, info.num_lanes
    nw = nc * ns
    n = b * t * h
    assert _CHUNK % lanes == 0 and t % _CHUNK == 0
    pairs = b * h
    assert pairs % nw == 0
    ppw = pairs // nw
    cpp = t // _CHUNK

    mesh = plsc.VectorSubcoreMesh(core_axis_name="c", subcore_axis_name="s")

    @functools.partial(
        pl.kernel,
        mesh=mesh,
        compiler_params=pltpu.CompilerParams(use_tc_tiling_on_sc=False,
                                             needs_layout_passes=False),
        out_type=jax.ShapeDtypeStruct((b, t, h, d), jnp.float32),
        scratch_types=[
            pltpu.VMEM((2, _CHUNK), jnp.int32),
            pltpu.VMEM((2, _CHUNK, d), jnp.float32),
            pltpu.VMEM((lanes,), jnp.int32),
            pltpu.SemaphoreType.DMA,
            pltpu.SemaphoreType.DMA,
        ],
    )
    def gather(ids_hbm, offs_hbm, table_hbm, out_hbm, idx_v, rows_v, offs_v,
               gsem, wsem):
        wid = lax.axis_index("s") * nc + lax.axis_index("c")
        pltpu.sync_copy(offs_hbm, offs_v)

        def pair_coords(pp):
            pair = wid * ppw + pp
            return pair // h, pair % h

        def load_shift(pp, c, slot):
            bi, hi = pair_coords(pp)
            pltpu.sync_copy(ids_hbm.at[bi, hi, pl.ds(c * _CHUNK, _CHUNK)],
                            idx_v.at[slot])
            off = plsc.load_gather(
                offs_v, [lax.broadcast(hi, (lanes,))])

            @pl.loop(0, _CHUNK // lanes)
            def _(i):
                sl = pl.ds(i * lanes, lanes)
                idx_v[slot, sl] = idx_v[slot, sl] + off

        def fire_gather(slot):
            return pltpu.async_copy(table_hbm.at[idx_v.at[slot]],
                                    rows_v.at[slot], gsem)

        def fire_write(pp, c, slot):
            bi, hi = pair_coords(pp)
            return pltpu.async_copy(
                rows_v.at[slot],
                out_hbm.at[bi, pl.ds(c * _CHUNK, _CHUNK), hi, :],
                wsem)

        n_chunks = ppw * cpp
        coords = [(pp, c) for pp in range(ppw) for c in range(cpp)]
        load_shift(*coords[0], 0)
        g = fire_gather(0)
        pending_w = []
        for k, (pp, c) in enumerate(coords):
            slot = k & 1
            if k + 1 < n_chunks:
                load_shift(*coords[k + 1], 1 - slot)
                if pending_w:
                    pending_w.pop(0).wait()
                g_next = fire_gather(1 - slot)
            g.wait()
            pending_w.append(fire_write(pp, c, slot))
            if k + 1 < n_chunks:
                g = g_next
        for w in pending_w:
            w.wait()

    return gather


@functools.lru_cache(maxsize=None)
def _build_tiler(n_rows, n_cols, rows_per_step):
    info = plsc.get_sparse_core_info()
    nw = info.num_cores * info.num_subcores
    nc = info.num_cores
    assert n_rows % (nw * rows_per_step) == 0 and n_cols % 128 == 0
    steps = n_rows // (nw * rows_per_step)
    per_w = n_rows // nw
    cpr = n_cols // 128
    in_rps = rows_per_step * cpr

    mesh = plsc.VectorSubcoreMesh(core_axis_name="c", subcore_axis_name="s")

    @functools.partial(
        pl.kernel,
        mesh=mesh,
        compiler_params=pltpu.CompilerParams(use_tc_tiling_on_sc=True),
        out_type=jax.ShapeDtypeStruct((n_rows, n_cols), jnp.float32),
        scratch_types=[
            pltpu.VMEM((2, in_rps, 128), jnp.float32),
            pltpu.SemaphoreType.DMA,
            pltpu.SemaphoreType.DMA,
        ],
    )
    def tile_out(in_hbm, out_hbm, buf, rsem, wsem):
        wid = lax.axis_index("s") * nc + lax.axis_index("c")
        row0 = wid * per_w

        def read(c, slot):
            return pltpu.async_copy(
                in_hbm.at[pl.ds((row0 + c * rows_per_step) * cpr, in_rps), :],
                buf.at[slot], rsem)

        r = read(0, 0)
        pending_w = []
        for c in range(steps):
            slot = c & 1
            if c + 1 < steps:
                if pending_w:
                    pending_w.pop(0).wait()
                r_next = read(c + 1, 1 - slot)
            r.wait()
            pending_w.append(pltpu.async_copy(
                buf.at[slot].reshape(rows_per_step, n_cols),
                out_hbm.at[pl.ds((row0 + c * rows_per_step), rows_per_step), :],
                wsem))
            if c + 1 < steps:
                r = r_next
        for w in pending_w:
            w.wait()

    return tile_out


def kernel(input_ids, offsets, table):
    b, t, h = input_ids.shape
    d = table.shape[1]
    n = b * t * h
    ids_bht = input_ids.transpose(0, 2, 1)
    out = _build(b, t, h, d)(ids_bht, offsets, table)
    out_tiled = _build_tiler(b * t, h * d, 16)(out.reshape(n * d // 128, 128))

    return out_tiled.reshape(b, t, h * d)
